# baseline (speedup 1.0000x reference)
import functools

import jax
import jax.numpy as jnp
from jax import lax
from jax.experimental import pallas as pl
from jax.experimental.pallas import tpu as pltpu
from jax.experimental.pallas import tpu_sc as plsc

EMBED = 64
NUM_CORES = 2
NUM_SUBCORES = 16
NUM_WORKERS = NUM_CORES * NUM_SUBCORES
CHUNK = 320


def _sc_gather_packed(flat_idx, table, num_indices):
    half = num_indices // 2
    per_w = half // NUM_WORKERS
    n_chunks = per_w // CHUNK
    mesh = plsc.VectorSubcoreMesh(core_axis_name="c", subcore_axis_name="s")

    @functools.partial(
        pl.kernel,
        out_type=jax.ShapeDtypeStruct((half, 2 * EMBED), table.dtype),
        mesh=mesh,
        scratch_types=[
            pltpu.VMEM((per_w,), jnp.int32),
            pltpu.VMEM((per_w,), jnp.int32),
            pltpu.VMEM((CHUNK, EMBED), jnp.float32),
            pltpu.VMEM((CHUNK, EMBED), jnp.float32),
            pltpu.VMEM((CHUNK, EMBED), jnp.float32),
            pltpu.VMEM((CHUNK, EMBED), jnp.float32),
            pltpu.SemaphoreType.DMA,
            pltpu.SemaphoreType.DMA,
            pltpu.SemaphoreType.DMA,
            pltpu.SemaphoreType.DMA,
            pltpu.SemaphoreType.DMA,
            pltpu.SemaphoreType.DMA,
            pltpu.SemaphoreType.DMA,
            pltpu.SemaphoreType.DMA,
        ],
        compiler_params=pltpu.CompilerParams(use_tc_tiling_on_sc=False),
    )
    def kfn(table_hbm, idx_hbm, out_hbm, idxL, idxR, rL0, rL1, rR0, rR1,
            sgL0, sgL1, sgR0, sgR1, soL0, soL1, soR0, soR1):
        wid = lax.axis_index("s") * NUM_CORES + lax.axis_index("c")
        base = wid * per_w
        rL, rR = (rL0, rL1), (rR0, rR1)
        sgL, sgR = (sgL0, sgL1), (sgR0, sgR1)
        soL, soR = (soL0, soL1), (soR0, soR1)

        pltpu.sync_copy(idx_hbm.at[pl.ds(base, per_w)], idxL)
        pltpu.sync_copy(idx_hbm.at[pl.ds(half + base, per_w)], idxR)

        def gather(j, b, idx_all, rows, sems):
            return pltpu.make_async_copy(
                table_hbm.at[idx_all.at[pl.ds(j * CHUNK, CHUNK)]],
                rows[b], sems[b])

        def writeback(j, b, rows, sems, lane0):
            return pltpu.make_async_copy(
                rows[b],
                out_hbm.at[pl.ds(base + j * CHUNK, CHUNK),
                           pl.ds(lane0, EMBED)],
                sems[b])

        for j in range(n_chunks):
            b = j % 2
            if j >= 2:
                writeback(j - 2, b, rL, soL, 0).wait()
                writeback(j - 2, b, rR, soR, EMBED).wait()
            gather(j, b, idxL, rL, sgL).start()
            gather(j, b, idxR, rR, sgR).start()
            if j >= 1:
                gather(j - 1, 1 - b, idxL, rL, sgL).wait()
                gather(j - 1, 1 - b, idxR, rR, sgR).wait()
                writeback(j - 1, 1 - b, rL, soL, 0).start()
                writeback(j - 1, 1 - b, rR, soR, EMBED).start()
        last = n_chunks - 1
        lb = last % 2
        gather(last, lb, idxL, rL, sgL).wait()
        gather(last, lb, idxR, rR, sgR).wait()
        writeback(last, lb, rL, soL, 0).start()
        writeback(last, lb, rR, soR, EMBED).start()
        writeback(last - 1, 1 - lb, rL, soL, 0).wait()
        writeback(last - 1, 1 - lb, rR, soR, EMBED).wait()
        writeback(last, lb, rL, soL, 0).wait()
        writeback(last, lb, rR, soR, EMBED).wait()

    return kfn(table, flat_idx)


def _tc_repack(packed, batch, seq):
    half_batch = batch // 2
    G = 16

    def body(in_ref, o_ref):
        for i in range(G):
            rows = pl.ds(i * seq, seq)
            o_ref[0, i] = in_ref[rows, pl.ds(0, EMBED)]
            o_ref[1, i] = in_ref[rows, pl.ds(EMBED, EMBED)]

    return pl.pallas_call(
        body,
        grid=(half_batch // G,),
        in_specs=[pl.BlockSpec((G * seq, 2 * EMBED), lambda g: (g, 0))],
        out_specs=pl.BlockSpec((2, G, seq, EMBED), lambda g: (0, g, 0, 0)),
        out_shape=jax.ShapeDtypeStruct((2, half_batch, seq, EMBED),
                                       jnp.float32),
    )(packed)


def kernel(x, table):
    batch, seq = x.shape
    num_indices = batch * seq
    flat_idx = x.reshape(num_indices).astype(jnp.int32)
    packed = _sc_gather_packed(flat_idx, table, num_indices)
    out = _tc_repack(packed, batch, seq)
    return out.reshape(batch, seq, EMBED)


# speedup vs baseline: 1.1456x; 1.1456x over previous
import functools

import jax
import jax.numpy as jnp
from jax import lax
from jax.experimental import pallas as pl
from jax.experimental.pallas import tpu as pltpu
from jax.experimental.pallas import tpu_sc as plsc

EMBED = 64
NUM_CORES = 2
NUM_SUBCORES = 16
NUM_WORKERS = NUM_CORES * NUM_SUBCORES
CHUNK = 320


def _sc_gather_packed(idx_even, idx_odd, table, half):
    per_w = half // NUM_WORKERS
    n_chunks = per_w // CHUNK
    mesh = plsc.VectorSubcoreMesh(core_axis_name="c", subcore_axis_name="s")

    @functools.partial(
        pl.kernel,
        out_type=jax.ShapeDtypeStruct((half, 2 * EMBED), table.dtype),
        mesh=mesh,
        scratch_types=[
            pltpu.VMEM((per_w,), jnp.int32),
            pltpu.VMEM((per_w,), jnp.int32),
            pltpu.VMEM((CHUNK, EMBED), jnp.float32),
            pltpu.VMEM((CHUNK, EMBED), jnp.float32),
            pltpu.VMEM((CHUNK, EMBED), jnp.float32),
            pltpu.VMEM((CHUNK, EMBED), jnp.float32),
            pltpu.SemaphoreType.DMA,
            pltpu.SemaphoreType.DMA,
            pltpu.SemaphoreType.DMA,
            pltpu.SemaphoreType.DMA,
            pltpu.SemaphoreType.DMA,
            pltpu.SemaphoreType.DMA,
            pltpu.SemaphoreType.DMA,
            pltpu.SemaphoreType.DMA,
        ],
        compiler_params=pltpu.CompilerParams(use_tc_tiling_on_sc=False),
    )
    def kfn(table_hbm, ie_hbm, io_hbm, out_hbm, idxE, idxO, rE0, rE1, rO0, rO1,
            sgE0, sgE1, sgO0, sgO1, soE0, soE1, soO0, soO1):
        wid = lax.axis_index("s") * NUM_CORES + lax.axis_index("c")
        base = wid * per_w
        rE, rO = (rE0, rE1), (rO0, rO1)
        sgE, sgO = (sgE0, sgE1), (sgO0, sgO1)
        soE, soO = (soE0, soE1), (soO0, soO1)

        pltpu.sync_copy(ie_hbm.at[pl.ds(base, per_w)], idxE)
        pltpu.sync_copy(io_hbm.at[pl.ds(base, per_w)], idxO)

        def gather(j, b, idx_all, rows, sems):
            return pltpu.make_async_copy(
                table_hbm.at[idx_all.at[pl.ds(j * CHUNK, CHUNK)]],
                rows[b], sems[b])

        def writeback(j, b, rows, sems, lane0):
            return pltpu.make_async_copy(
                rows[b],
                out_hbm.at[pl.ds(base + j * CHUNK, CHUNK),
                           pl.ds(lane0, EMBED)],
                sems[b])

        for j in range(n_chunks):
            b = j % 2
            if j >= 2:
                writeback(j - 2, b, rE, soE, 0).wait()
                writeback(j - 2, b, rO, soO, EMBED).wait()
            gather(j, b, idxE, rE, sgE).start()
            gather(j, b, idxO, rO, sgO).start()
            if j >= 1:
                gather(j - 1, 1 - b, idxE, rE, sgE).wait()
                gather(j - 1, 1 - b, idxO, rO, sgO).wait()
                writeback(j - 1, 1 - b, rE, soE, 0).start()
                writeback(j - 1, 1 - b, rO, soO, EMBED).start()
        last = n_chunks - 1
        lb = last % 2
        gather(last, lb, idxE, rE, sgE).wait()
        gather(last, lb, idxO, rO, sgO).wait()
        writeback(last, lb, rE, soE, 0).start()
        writeback(last, lb, rO, soO, EMBED).start()
        writeback(last - 1, 1 - lb, rE, soE, 0).wait()
        writeback(last - 1, 1 - lb, rO, soO, EMBED).wait()
        writeback(last, lb, rE, soE, 0).wait()
        writeback(last, lb, rO, soO, EMBED).wait()

    return kfn(table, idx_even, idx_odd)


def _tc_transpose(packed, batch, seq):
    def body(in_ref, o_ref):
        o_ref[0] = in_ref[:, 0:EMBED].T
        o_ref[1] = in_ref[:, EMBED:2 * EMBED].T

    return pl.pallas_call(
        body,
        grid=(seq // 2,),
        in_specs=[pl.BlockSpec((batch, 2 * EMBED), lambda u: (u, 0))],
        out_specs=pl.BlockSpec((2, EMBED, batch), lambda u: (u, 0, 0)),
        out_shape=jax.ShapeDtypeStruct((seq, EMBED, batch), jnp.float32),
    )(packed)


def kernel(x, table):
    batch, seq = x.shape
    num_indices = batch * seq
    xt = x.T.astype(jnp.int32)
    idx_even = xt[0::2].reshape(num_indices // 2)
    idx_odd = xt[1::2].reshape(num_indices // 2)
    packed = _sc_gather_packed(idx_even, idx_odd, table, num_indices // 2)
    out = _tc_transpose(packed, batch, seq)
    return out.transpose(2, 0, 1)


# speedup vs baseline: 1.8549x; 1.6191x over previous
import functools

import jax
import jax.numpy as jnp
from jax import lax
from jax.experimental import pallas as pl
from jax.experimental.pallas import tpu as pltpu
from jax.experimental.pallas import tpu_sc as plsc

EMBED = 64
NUM_CORES = 2
NUM_SUBCORES = 16
NUM_WORKERS = NUM_CORES * NUM_SUBCORES
CHUNK = 320


PACK_BLK = 6400
PACK_V1 = 524800
PACK_SHIFT = 480000


def _tc_pack_table(table_t):
    def body(a_ref, b_ref, o_ref):
        o_ref[:, 0:EMBED] = a_ref[...].T
        o_ref[:, EMBED:2 * EMBED] = b_ref[...].T

    return pl.pallas_call(
        body,
        grid=(PACK_V1 // PACK_BLK,),
        in_specs=[
            pl.BlockSpec((EMBED, PACK_BLK), lambda g: (0, g)),
            pl.BlockSpec((EMBED, PACK_BLK),
                         lambda g: (0, g + PACK_SHIFT // PACK_BLK)),
        ],
        out_specs=pl.BlockSpec((PACK_BLK, 2 * EMBED), lambda g: (g, 0)),
        out_shape=jax.ShapeDtypeStruct((PACK_V1, 2 * EMBED), jnp.float32),
        compiler_params=pltpu.CompilerParams(
            dimension_semantics=("parallel",)),
    )(table_t, table_t)


def _sc_gather(idx_row, packed_table, num_indices):
    per_w = num_indices // NUM_WORKERS
    n_chunks = per_w // CHUNK
    mesh = plsc.VectorSubcoreMesh(core_axis_name="c", subcore_axis_name="s")

    @functools.partial(
        pl.kernel,
        out_type=jax.ShapeDtypeStruct((num_indices, 2 * EMBED), jnp.float32),
        mesh=mesh,
        scratch_types=[
            pltpu.VMEM((per_w,), jnp.int32),
            pltpu.VMEM((CHUNK, 2 * EMBED), jnp.float32),
            pltpu.VMEM((CHUNK, 2 * EMBED), jnp.float32),
            pltpu.SemaphoreType.DMA,
            pltpu.SemaphoreType.DMA,
            pltpu.SemaphoreType.DMA,
            pltpu.SemaphoreType.DMA,
        ],
        compiler_params=pltpu.CompilerParams(use_tc_tiling_on_sc=False),
    )
    def kfn(tab_hbm, idx_hbm, out_hbm, idx_all, r0, r1, sg0, sg1, so0, so1):
        wid = lax.axis_index("s") * NUM_CORES + lax.axis_index("c")
        base = wid * per_w
        rows = (r0, r1)
        sg = (sg0, sg1)
        so = (so0, so1)

        pltpu.sync_copy(idx_hbm.at[pl.ds(base, per_w)], idx_all)

        def gather(j, b):
            return pltpu.make_async_copy(
                tab_hbm.at[idx_all.at[pl.ds(j * CHUNK, CHUNK)]],
                rows[b], sg[b])

        def writeback(j, b):
            return pltpu.make_async_copy(
                rows[b], out_hbm.at[pl.ds(base + j * CHUNK, CHUNK)], so[b])

        gather(0, 0).start()
        for j in range(1, n_chunks):
            b = j % 2
            if j >= 2:
                writeback(j - 2, b).wait()
            gather(j, b).start()
            gather(j - 1, 1 - b).wait()
            writeback(j - 1, 1 - b).start()
        last = n_chunks - 1
        gather(last, last % 2).wait()
        writeback(last, last % 2).start()
        writeback(last - 1, (last - 1) % 2).wait()
        writeback(last, last % 2).wait()

    return kfn(packed_table, idx_row)


def _tc_select_transpose(inter, x_t, batch, seq):
    def body(in_ref, x_ref, o_ref):
        s = pl.program_id(0)
        hi = x_ref[pl.ds(s, 1)][0] >= PACK_V1
        left = in_ref[:, 0:EMBED].T
        right = in_ref[:, EMBED:2 * EMBED].T
        o_ref[0] = jnp.where(hi[None, :], right, left)

    return pl.pallas_call(
        body,
        grid=(seq,),
        in_specs=[
            pl.BlockSpec((batch, 2 * EMBED), lambda s: (s, 0)),
            pl.BlockSpec((seq, batch), lambda s: (0, 0)),
        ],
        out_specs=pl.BlockSpec((1, EMBED, batch), lambda s: (s, 0, 0)),
        out_shape=jax.ShapeDtypeStruct((seq, EMBED, batch), jnp.float32),
        compiler_params=pltpu.CompilerParams(
            dimension_semantics=("parallel",)),
    )(inter, x_t)


def kernel(x, table):
    batch, seq = x.shape
    num_indices = batch * seq
    packed_table = _tc_pack_table(table.T)
    x_t = x.T.astype(jnp.int32)
    flat_t = x_t.reshape(num_indices)
    idx_row = jnp.where(flat_t < PACK_V1, flat_t, flat_t - PACK_SHIFT)
    inter = _sc_gather(idx_row, packed_table, num_indices)
    out = _tc_select_transpose(inter, x_t, batch, seq)
    return out.transpose(2, 0, 1)
